# baseline (speedup 1.0000x reference)
import functools

import jax
import jax.numpy as jnp
from jax import lax
from jax.experimental import pallas as pl
from jax.experimental.pallas import tpu as pltpu
from jax.experimental.pallas import tpu_sc as plsc


def _sc_gather(adj, rem_tab, idx, width):
    b = idx.shape[0]
    pad_w = rem_tab.shape[1]
    info = plsc.get_sparse_core_info()
    nw = info.num_cores * info.num_subcores
    b_per_w = b // nw
    chunk = 8
    n_chunks = b_per_w // chunk

    mesh = plsc.VectorSubcoreMesh(core_axis_name="c", subcore_axis_name="s")

    @functools.partial(
        pl.kernel,
        mesh=mesh,
        out_type=jax.ShapeDtypeStruct((b, width + pad_w), jnp.float32),
        scratch_types=[
            pltpu.VMEM((chunk,), jnp.int32),
            pltpu.VMEM((chunk,), jnp.int32),
            pltpu.VMEM((chunk, width), jnp.float32),
            pltpu.VMEM((chunk, pad_w), jnp.float32),
            pltpu.SemaphoreType.DMA,
            pltpu.SemaphoreType.DMA,
        ],
    )
    def gather_kernel(adj_hbm, tab_hbm, idx_hbm, idx2_hbm, out_hbm,
                      idx_v, idx2_v, rows_v, rem_v, sem, sem2):
        wid = lax.axis_index("s") * info.num_cores + lax.axis_index("c")
        base = wid * b_per_w

        def body(c, carry):
            row0 = pl.multiple_of(base + c * chunk, chunk)
            pltpu.sync_copy(idx_hbm.at[pl.ds(row0, chunk)], idx_v)
            pltpu.sync_copy(idx2_hbm.at[pl.ds(row0, chunk)], idx2_v)
            cp1 = pltpu.async_copy(adj_hbm.at[idx_v, pl.ds(0, width)],
                                   rows_v, sem)
            cp2 = pltpu.async_copy(tab_hbm.at[idx2_v], rem_v, sem2)
            cp1.wait()
            cp2.wait()
            pltpu.sync_copy(rows_v,
                            out_hbm.at[pl.ds(row0, chunk), pl.ds(0, width)])
            pltpu.sync_copy(rem_v,
                            out_hbm.at[pl.ds(row0, chunk),
                                       pl.ds(width, pad_w)])
            return carry

        lax.fori_loop(0, n_chunks, body, 0)

    return gather_kernel(adj, rem_tab, idx, idx // 8)


def _layer1_body(x_ref, w1_ref, adj_ref, b1_ref, w2_ref, o_ref, xw1_s):
    @pl.when(pl.program_id(0) == 0)
    def _():
        xw1_s[...] = jnp.dot(x_ref[...], w1_ref[...],
                             preferred_element_type=jnp.float32)

    acc = jnp.dot(adj_ref[...], xw1_s[...],
                  preferred_element_type=jnp.float32)
    h = jnp.maximum(acc + b1_ref[...], 0.0)
    o_ref[...] = jnp.dot(h, w2_ref[...], preferred_element_type=jnp.float32)


def _layer1(adj, x, W1, b1, W2, bm=400):
    n = adj.shape[0]
    nfeat = x.shape[1]
    nhid = W1.shape[1]
    ncls = W2.shape[1]
    grid = (n // bm,)
    return pl.pallas_call(
        _layer1_body,
        grid=grid,
        in_specs=[
            pl.BlockSpec((n, nfeat), lambda i: (0, 0)),
            pl.BlockSpec((nfeat, nhid), lambda i: (0, 0)),
            pl.BlockSpec((bm, n), lambda i: (i, 0)),
            pl.BlockSpec((1, nhid), lambda i: (0, 0)),
            pl.BlockSpec((nhid, ncls), lambda i: (0, 0)),
        ],
        out_specs=pl.BlockSpec((bm, ncls), lambda i: (i, 0)),
        out_shape=jax.ShapeDtypeStruct((n, ncls), jnp.float32),
        scratch_shapes=[pltpu.VMEM((n, nhid), jnp.float32)],
    )(x, W1, adj, b1.reshape(1, nhid), W2)


def _layer2_body(ai_ref, sel_ref, hw2_ref, b2_ref, o_ref, *, width, rem):
    sel = sel_ref[...]
    g = ai_ref[:, width:]
    gsel = jnp.zeros((g.shape[0], rem), jnp.float32)
    for k in range(128 // rem):
        gsel = gsel + jnp.where(sel == k, 1.0, 0.0) * g[:, k * rem:(k + 1) * rem]
    o = jnp.dot(ai_ref[:, :width], hw2_ref[:width],
                preferred_element_type=jnp.float32)
    o = o + jnp.dot(gsel, hw2_ref[width:],
                    preferred_element_type=jnp.float32)
    o = o + b2_ref[...]
    m = jnp.max(o, axis=1, keepdims=True)
    lse = jnp.log(jnp.sum(jnp.exp(o - m), axis=1, keepdims=True)) + m
    o_ref[...] = o - lse


def _layer2(adj_idx, idx_mod, hw2, b2, bm=256):
    b, tot_w = adj_idx.shape
    n, ncls = hw2.shape
    width = (n // 128) * 128
    rem = n - width
    grid = (b // bm,)
    return pl.pallas_call(
        functools.partial(_layer2_body, width=width, rem=rem),
        grid=grid,
        in_specs=[
            pl.BlockSpec((bm, tot_w), lambda i: (i, 0)),
            pl.BlockSpec((bm, 1), lambda i: (i, 0)),
            pl.BlockSpec((n, ncls), lambda i: (0, 0)),
            pl.BlockSpec((1, ncls), lambda i: (0, 0)),
        ],
        out_specs=pl.BlockSpec((bm, ncls), lambda i: (i, 0)),
        out_shape=jax.ShapeDtypeStruct((b, ncls), jnp.float32),
    )(adj_idx, idx_mod.reshape(b, 1), hw2, b2.reshape(1, ncls))


def kernel(x, adj, idx, W1, b1, W2, b2):
    n = adj.shape[0]
    width = (n // 128) * 128
    rem = n - width
    idx = idx.astype(jnp.int32)
    rem_tab = lax.slice(adj, (0, width), (n, n)).reshape(n * rem // 128, 128)
    adj_idx = _sc_gather(adj, rem_tab, idx, width)
    hw2 = _layer1(adj, x, W1, b1, W2)
    return _layer2(adj_idx, idx % 8, hw2, b2)


# speedup vs baseline: 1.0107x; 1.0107x over previous
import functools

import jax
import jax.numpy as jnp
from jax import lax
from jax.experimental import pallas as pl
from jax.experimental.pallas import tpu as pltpu
from jax.experimental.pallas import tpu_sc as plsc


def _sc_gather(adj, rem_tab, idx, width):
    b = idx.shape[0]
    pad_w = rem_tab.shape[1]
    info = plsc.get_sparse_core_info()
    nw = info.num_cores * info.num_subcores
    b_per_w = b // nw
    chunk = 8
    n_chunks = b_per_w // chunk

    mesh = plsc.VectorSubcoreMesh(core_axis_name="c", subcore_axis_name="s")

    @functools.partial(
        pl.kernel,
        mesh=mesh,
        out_type=jax.ShapeDtypeStruct((b, width + pad_w), jnp.float32),
        scratch_types=[
            pltpu.VMEM((chunk,), jnp.int32),
            pltpu.VMEM((chunk, width), jnp.float32),
            pltpu.VMEM((chunk, pad_w), jnp.float32),
            pltpu.SemaphoreType.DMA,
            pltpu.SemaphoreType.DMA,
        ],
    )
    def gather_kernel(adj_hbm, tab_hbm, idx_hbm, out_hbm,
                      idx_v, rows_v, rem_v, sem, sem2):
        wid = lax.axis_index("s") * info.num_cores + lax.axis_index("c")
        base = wid * b_per_w

        def body(c, carry):
            row0 = pl.multiple_of(base + c * chunk, chunk)
            pltpu.sync_copy(idx_hbm.at[pl.ds(row0, chunk)], idx_v)
            cp1 = pltpu.async_copy(adj_hbm.at[idx_v, pl.ds(0, width)],
                                   rows_v, sem)
            cp2 = pltpu.async_copy(tab_hbm.at[idx_v], rem_v, sem2)
            cp1.wait()
            cp2.wait()
            pltpu.sync_copy(rows_v,
                            out_hbm.at[pl.ds(row0, chunk), pl.ds(0, width)])
            pltpu.sync_copy(rem_v,
                            out_hbm.at[pl.ds(row0, chunk),
                                       pl.ds(width, pad_w)])
            return carry

        lax.fori_loop(0, n_chunks, body, 0)

    return gather_kernel(adj, rem_tab, idx)


def _layer1_body(x_ref, w1_ref, adj_ref, b1_ref, w2_ref, o_ref, xw1_s):
    @pl.when(pl.program_id(0) == 0)
    def _():
        xw1_s[...] = jnp.dot(x_ref[...], w1_ref[...],
                             preferred_element_type=jnp.float32)

    acc = jnp.dot(adj_ref[...], xw1_s[...],
                  preferred_element_type=jnp.float32)
    h = jnp.maximum(acc + b1_ref[...], 0.0)
    o_ref[...] = jnp.dot(h, w2_ref[...], preferred_element_type=jnp.float32)


def _layer1(adj, x, W1, b1, W2, bm=400):
    n = adj.shape[0]
    nfeat = x.shape[1]
    nhid = W1.shape[1]
    ncls = W2.shape[1]
    grid = (n // bm,)
    return pl.pallas_call(
        _layer1_body,
        grid=grid,
        in_specs=[
            pl.BlockSpec((n, nfeat), lambda i: (0, 0)),
            pl.BlockSpec((nfeat, nhid), lambda i: (0, 0)),
            pl.BlockSpec((bm, n), lambda i: (i, 0)),
            pl.BlockSpec((1, nhid), lambda i: (0, 0)),
            pl.BlockSpec((nhid, ncls), lambda i: (0, 0)),
        ],
        out_specs=pl.BlockSpec((bm, ncls), lambda i: (i, 0)),
        out_shape=jax.ShapeDtypeStruct((n, ncls), jnp.float32),
        scratch_shapes=[pltpu.VMEM((n, nhid), jnp.float32)],
    )(x, W1, adj, b1.reshape(1, nhid), W2)


def _layer2_body(ai_ref, hw2_ref, b2_ref, o_ref, *, width, rem):
    o = jnp.dot(ai_ref[:, :width], hw2_ref[:width],
                preferred_element_type=jnp.float32)
    o = o + jnp.dot(ai_ref[:, -rem:], hw2_ref[width:],
                    preferred_element_type=jnp.float32)
    o = o + b2_ref[...]
    m = jnp.max(o, axis=1, keepdims=True)
    lse = jnp.log(jnp.sum(jnp.exp(o - m), axis=1, keepdims=True)) + m
    o_ref[...] = o - lse


def _layer2(adj_idx, hw2, b2, bm=256):
    b, tot_w = adj_idx.shape
    n, ncls = hw2.shape
    width = (n // 128) * 128
    rem = n - width
    grid = (b // bm,)
    return pl.pallas_call(
        functools.partial(_layer2_body, width=width, rem=rem),
        grid=grid,
        in_specs=[
            pl.BlockSpec((bm, tot_w), lambda i: (i, 0)),
            pl.BlockSpec((n, ncls), lambda i: (0, 0)),
            pl.BlockSpec((1, ncls), lambda i: (0, 0)),
        ],
        out_specs=pl.BlockSpec((bm, ncls), lambda i: (i, 0)),
        out_shape=jax.ShapeDtypeStruct((b, ncls), jnp.float32),
    )(adj_idx, hw2, b2.reshape(1, ncls))


def kernel(x, adj, idx, W1, b1, W2, b2):
    n = adj.shape[0]
    width = (n // 128) * 128
    rem = n - width
    idx = idx.astype(jnp.int32)
    rem_tab = jnp.pad(lax.slice(adj, (0, width), (n, n)),
                      ((0, 0), (128 - rem, 0)))
    adj_idx = _sc_gather(adj, rem_tab, idx, width)
    hw2 = _layer1(adj, x, W1, b1, W2)
    return _layer2(adj_idx, hw2, b2)
